# baseline (speedup 1.0000x reference)
import functools

import numpy as np
import jax
import jax.numpy as jnp
from jax import lax
from jax.experimental import pallas as pl
from jax.experimental.pallas import tpu as pltpu
from jax.experimental.pallas import tpu_sc as plsc

SEQ = 200
DIM = 64
LANES = 16
NC = 2
NS = 16
NW = NC * NS

CHUNK = 400
NBUF = 3
SUBS = [(o, min(128, CHUNK - o)) for o in range(0, CHUNK, 128)]


def _pos_table_np() -> np.ndarray:
    pos = np.arange(SEQ, dtype=np.float64)[:, None]
    emb = np.arange(DIM, dtype=np.float64)[None, :]
    tmp = pos / (10000.0 ** (2.0 * emb / DIM))
    even_len = DIM // 2 + DIM % 2
    odd_len = DIM // 2
    out = np.zeros((SEQ, DIM), dtype=np.float64)
    out[:, 0::2] = np.sin(tmp)[:, :even_len]
    out[:, 1::2] = np.cos(tmp)[:, :odd_len]
    return out.astype(np.float32)


_POS = _pos_table_np()


@functools.partial(jax.jit, static_argnames=("total_rows",))
def _lookup(table, idx, pos, *, total_rows):
    assert total_rows % (NW * CHUNK) == 0
    bpw = total_rows // NW
    nchunk = bpw // CHUNK
    assert (nchunk - 4) % 6 == 0

    mesh = plsc.VectorSubcoreMesh(core_axis_name="c", subcore_axis_name="s")

    @functools.partial(
        pl.kernel,
        mesh=mesh,
        out_type=jax.ShapeDtypeStruct((total_rows // SEQ, SEQ, DIM),
                                      jnp.float32),
        compiler_params=pltpu.CompilerParams(
            use_tc_tiling_on_sc=False, skip_device_barrier=True),
        scratch_types=[
            pltpu.VMEM((CHUNK,), jnp.int32),
            pltpu.VMEM((CHUNK,), jnp.int32),
            pltpu.VMEM((SEQ, DIM), jnp.float32),
            pltpu.VMEM((CHUNK // SEQ, SEQ, DIM), jnp.float32),
            pltpu.VMEM((CHUNK // SEQ, SEQ, DIM), jnp.float32),
            pltpu.VMEM((CHUNK // SEQ, SEQ, DIM), jnp.float32),
            pltpu.SemaphoreType.DMA,
            pltpu.SemaphoreType.DMA,
            pltpu.SemaphoreType.DMA,
        ],
    )
    def body(table_hbm, idx_hbm, pos_hbm, out_hbm,
             idx_v0, idx_v1, pos_v, r0, r1, r2,
             sem_g, sem_ix, sem_s):
        wid = lax.axis_index("s") * NC + lax.axis_index("c")
        base = wid * bpw
        idx_b = (idx_v0, idx_v1)
        rows_b = (r0, r1, r2)

        def idx_copy(m, ib):
            return pltpu.make_async_copy(
                idx_hbm.at[pl.ds(base + m * CHUNK, CHUNK)], idx_b[ib], sem_ix)

        def gather_copies(ib, rb):
            cps = []
            for h in range(CHUNK // SEQ):
                for (o, n) in ((0, 128), (128, SEQ - 128)):
                    cps.append(pltpu.make_async_copy(
                        table_hbm.at[idx_b[ib].at[pl.ds(h * SEQ + o, n)]],
                        rows_b[rb].at[h, pl.ds(o, n)], sem_g))
            return cps

        nbat = CHUNK // SEQ
        bbase = wid * (bpw // SEQ)

        def scatter_copy(m, rb):
            return pltpu.make_async_copy(
                rows_b[rb], out_hbm.at[pl.ds(bbase + m * nbat, nbat)], sem_s)

        def add_pos(rb):
            rows = rows_b[rb]

            @plsc.parallel_loop(0, SEQ, step=1, unroll=4)
            def _pbody(p):
                for j in range(DIM // LANES):
                    pv = pos_v[p, pl.ds(j * LANES, LANES)]
                    for c in range(CHUNK // SEQ):
                        plsc.addupdate(
                            rows.at[c, p, pl.ds(j * LANES, LANES)], pv)

        def one(i, rb, ib, wait_s, next_g, next_ix):
            for cp in gather_copies(ib, rb):
                cp.wait()
            if wait_s:
                scatter_copy(i, (rb + 1) % NBUF).wait()
            if next_g:
                idx_copy(i + 1, ib ^ 1).wait()
                for cp in gather_copies(ib ^ 1, (rb + 1) % NBUF):
                    cp.start()
            if next_ix:
                idx_copy(i + 2, ib).start()
            add_pos(rb)
            scatter_copy(i, rb).start()

        pltpu.sync_copy(pos_hbm, pos_v)
        pltpu.sync_copy(idx_hbm.at[pl.ds(base, CHUNK)], idx_v0)
        for cp in gather_copies(0, 0):
            cp.start()
        idx_copy(1, 1).start()

        one(0, 0, 0, False, True, True)
        one(1, 1, 1, False, True, True)

        def mid(g, carry):
            i0 = 2 + 6 * g
            for k in range(6):
                one(i0 + k, (2 + k) % 3, k % 2, True, True, True)
            return carry
        lax.fori_loop(0, (nchunk - 4) // 6, mid, 0)

        one(nchunk - 2, (nchunk - 2) % 3, (nchunk - 2) % 2, True, True, False)
        one(nchunk - 1, (nchunk - 1) % 3, (nchunk - 1) % 2, True, False, False)

        scatter_copy(nchunk - 2, (nchunk - 2) % 3).wait()
        scatter_copy(nchunk - 1, (nchunk - 1) % 3).wait()

    return body(table, idx, pos)


def kernel(x, embeddings):
    b, s = x.shape
    idx = x.reshape(-1).astype(jnp.int32)
    pos = jnp.asarray(_POS)
    return _lookup(embeddings, idx, pos, total_rows=b * s)


# speedup vs baseline: 1.3316x; 1.3316x over previous
import functools

import numpy as np
import jax
import jax.numpy as jnp
from jax import lax
from jax.experimental import pallas as pl
from jax.experimental.pallas import tpu as pltpu
from jax.experimental.pallas import tpu_sc as plsc

SEQ = 200
DIM = 64
LANES = 16
NC = 2
NS = 16
NW = NC * NS

CHUNK = 400
NBUF = 3
SUBS = [(o, min(128, CHUNK - o)) for o in range(0, CHUNK, 128)]


def _pos_table_np() -> np.ndarray:
    pos = np.arange(SEQ, dtype=np.float64)[:, None]
    emb = np.arange(DIM, dtype=np.float64)[None, :]
    tmp = pos / (10000.0 ** (2.0 * emb / DIM))
    even_len = DIM // 2 + DIM % 2
    odd_len = DIM // 2
    out = np.zeros((SEQ, DIM), dtype=np.float64)
    out[:, 0::2] = np.sin(tmp)[:, :even_len]
    out[:, 1::2] = np.cos(tmp)[:, :odd_len]
    return out.astype(np.float32)


_POS = _pos_table_np()


@functools.partial(jax.jit, static_argnames=("total_rows",))
def _lookup(table, idx, pos, *, total_rows):
    assert total_rows % (NW * CHUNK) == 0
    bpw = total_rows // NW
    nchunk = bpw // CHUNK
    assert (nchunk - 4) % 6 == 0

    mesh = plsc.VectorSubcoreMesh(core_axis_name="c", subcore_axis_name="s")

    @functools.partial(
        pl.kernel,
        mesh=mesh,
        out_type=jax.ShapeDtypeStruct((total_rows // SEQ, SEQ, 2 * DIM),
                                      jnp.float32),
        compiler_params=pltpu.CompilerParams(
            use_tc_tiling_on_sc=False, skip_device_barrier=True),
        scratch_types=[
            pltpu.VMEM((CHUNK,), jnp.int32),
            pltpu.VMEM((CHUNK,), jnp.int32),
            pltpu.VMEM((SEQ, DIM), jnp.float32),
            pltpu.VMEM((CHUNK // SEQ, SEQ, DIM), jnp.float32),
            pltpu.VMEM((CHUNK // SEQ, SEQ, DIM), jnp.float32),
            pltpu.VMEM((CHUNK // SEQ, SEQ, DIM), jnp.float32),
            pltpu.SemaphoreType.DMA,
            pltpu.SemaphoreType.DMA,
            pltpu.SemaphoreType.DMA,
        ],
    )
    def body(table_hbm, idx_hbm, pos_hbm, out_hbm,
             idx_v0, idx_v1, pos_v, r0, r1, r2,
             sem_g, sem_ix, sem_s):
        wid = lax.axis_index("s") * NC + lax.axis_index("c")
        base = wid * bpw
        idx_b = (idx_v0, idx_v1)
        rows_b = (r0, r1, r2)

        def idx_copy(m, ib):
            return pltpu.make_async_copy(
                idx_hbm.at[pl.ds(base + m * CHUNK, CHUNK)], idx_b[ib], sem_ix)

        def gather_copies(ib, rb):
            cps = []
            for h in range(CHUNK // SEQ):
                for (o, n) in ((0, 128), (128, SEQ - 128)):
                    cps.append(pltpu.make_async_copy(
                        table_hbm.at[idx_b[ib].at[pl.ds(h * SEQ + o, n)]],
                        rows_b[rb].at[h, pl.ds(o, n)], sem_g))
            return cps

        nbat = CHUNK // SEQ
        bbase = wid * (bpw // SEQ)

        def scatter_copy(m, rb):
            return pltpu.make_async_copy(
                rows_b[rb],
                out_hbm.at[pl.ds(bbase + m * nbat, nbat), :, pl.ds(0, DIM)],
                sem_s)

        def add_pos(rb):
            rows = rows_b[rb]

            @plsc.parallel_loop(0, SEQ, step=1, unroll=4)
            def _pbody(p):
                for j in range(DIM // LANES):
                    pv = pos_v[p, pl.ds(j * LANES, LANES)]
                    for c in range(CHUNK // SEQ):
                        plsc.addupdate(
                            rows.at[c, p, pl.ds(j * LANES, LANES)], pv)

        def one(i, rb, ib, wait_s, next_g, next_ix):
            for cp in gather_copies(ib, rb):
                cp.wait()
            if wait_s:
                scatter_copy(i, (rb + 1) % NBUF).wait()
            if next_g:
                idx_copy(i + 1, ib ^ 1).wait()
                for cp in gather_copies(ib ^ 1, (rb + 1) % NBUF):
                    cp.start()
            if next_ix:
                idx_copy(i + 2, ib).start()
            add_pos(rb)
            scatter_copy(i, rb).start()

        pltpu.sync_copy(pos_hbm, pos_v)
        pltpu.sync_copy(idx_hbm.at[pl.ds(base, CHUNK)], idx_v0)
        for cp in gather_copies(0, 0):
            cp.start()
        idx_copy(1, 1).start()

        one(0, 0, 0, False, True, True)
        one(1, 1, 1, False, True, True)

        def mid(g, carry):
            i0 = 2 + 6 * g
            for k in range(6):
                one(i0 + k, (2 + k) % 3, k % 2, True, True, True)
            return carry
        lax.fori_loop(0, (nchunk - 4) // 6, mid, 0)

        one(nchunk - 2, (nchunk - 2) % 3, (nchunk - 2) % 2, True, True, False)
        one(nchunk - 1, (nchunk - 1) % 3, (nchunk - 1) % 2, True, False, False)

        scatter_copy(nchunk - 2, (nchunk - 2) % 3).wait()
        scatter_copy(nchunk - 1, (nchunk - 1) % 3).wait()

    return body(table, idx, pos)


def kernel(x, embeddings):
    b, s = x.shape
    idx = x.reshape(-1).astype(jnp.int32)
    pos = jnp.asarray(_POS)
    out128 = _lookup(embeddings, idx, pos, total_rows=b * s)
    return out128[:, :, :DIM]
